# baseline (speedup 1.0000x reference)
import functools

import jax
import jax.numpy as jnp
from jax.experimental import pallas as pl

N = 10000
IN_FEAT = 128
OUT_FEAT = 16
BI = 400


def _proj_kernel(feat_ref, w_ref, h_ref):
    h_ref[...] = jnp.dot(feat_ref[...], w_ref[...],
                         preferred_element_type=jnp.float32)


def _spmm_kernel(adj_ref, h_ref, out_ref):
    out_ref[...] = jnp.dot(adj_ref[...], h_ref[...],
                           preferred_element_type=jnp.float32)


@jax.jit
def kernel(feat, adj, W):
    h = pl.pallas_call(
        _proj_kernel,
        out_shape=jax.ShapeDtypeStruct((N, OUT_FEAT), jnp.float32),
    )(feat, W)

    out = pl.pallas_call(
        _spmm_kernel,
        grid=(N // BI,),
        in_specs=[
            pl.BlockSpec((BI, N), lambda i: (i, 0)),
            pl.BlockSpec((N, OUT_FEAT), lambda i: (0, 0)),
        ],
        out_specs=pl.BlockSpec((BI, OUT_FEAT), lambda i: (i, 0)),
        out_shape=jax.ShapeDtypeStruct((N, OUT_FEAT), jnp.float32),
    )(adj, h)
    return (out, feat.shape[0] - 1)


# speedup vs baseline: 1.0636x; 1.0636x over previous
import jax
import jax.numpy as jnp
from jax.experimental import pallas as pl
from jax.experimental.pallas import tpu as pltpu

N = 10000
IN_FEAT = 128
OUT_FEAT = 16
BI = 400


def _fused_kernel(feat_ref, w_ref, adj_ref, out_ref, h_ref):
    @pl.when(pl.program_id(0) == 0)
    def _():
        h_ref[...] = jnp.dot(feat_ref[...], w_ref[...],
                             preferred_element_type=jnp.float32)

    out_ref[...] = jnp.dot(adj_ref[...], h_ref[...],
                           preferred_element_type=jnp.float32)


@jax.jit
def kernel(feat, adj, W):
    out = pl.pallas_call(
        _fused_kernel,
        grid=(N // BI,),
        in_specs=[
            pl.BlockSpec((N, IN_FEAT), lambda i: (0, 0)),
            pl.BlockSpec((IN_FEAT, OUT_FEAT), lambda i: (0, 0)),
            pl.BlockSpec((BI, N), lambda i: (i, 0)),
        ],
        out_specs=pl.BlockSpec((BI, OUT_FEAT), lambda i: (i, 0)),
        out_shape=jax.ShapeDtypeStruct((N, OUT_FEAT), jnp.float32),
        scratch_shapes=[pltpu.VMEM((N, OUT_FEAT), jnp.float32)],
    )(feat, W, adj)
    return (out, feat.shape[0] - 1)
